# Initial kernel scaffold; baseline (speedup 1.0000x reference)
import jax
import jax.numpy as jnp
from jax.experimental import pallas as pl


def kernel(h, edge_core, core_edge_idx, Watt, batt, Wnode, bnode, W1, b1, W2, b2, Wh):
    raise NotImplementedError("write your pallas kernel here")


# speedup vs baseline: 3.8516x; 3.8516x over previous
import functools

import jax
import jax.numpy as jnp
from jax import lax
from jax.experimental import pallas as pl
from jax.experimental.pallas import tpu as pltpu
from jax.experimental.pallas import tpu_sc as plsc

N = 10000
E = 320000
D_NODE = 128
D_EDGE = 16
H = 20
ATT_EMB = 1280
D_HEAD = ATT_EMB // H
D_FF = ATT_EMB * 4
INV_SQRT_DHEAD = 1.0 / (D_HEAD ** 0.5)

_GB = 128
_NB = (2 * E) // _GB
_NW = 32
_NG = _NB // 8
_PERG = _NG // _NW
_REMG = _NG - _PERG * _NW
_MAXROWS = (_PERG + 1) * 8


def _sc_gather(h, idx2d):
    mesh = plsc.VectorSubcoreMesh(core_axis_name="c", subcore_axis_name="s")

    @functools.partial(
        pl.kernel,
        mesh=mesh,
        out_type=jax.ShapeDtypeStruct((2 * E, D_NODE), jnp.float32),
        scratch_types=[
            pltpu.VMEM((_MAXROWS, _GB), jnp.int32),
            pltpu.VMEM((_GB, D_NODE), jnp.float32),
            pltpu.SemaphoreType.DMA,
        ],
    )
    def gather_kernel(h_hbm, idx_hbm, out_hbm, idx_v, rows_v, sem):
        c = lax.axis_index("c")
        s = lax.axis_index("s")
        wid = s * 2 + c
        ng = jnp.where(wid < _REMG, _PERG + 1, _PERG)
        startg = jnp.where(
            wid < _REMG, wid * (_PERG + 1),
            _REMG * (_PERG + 1) + (wid - _REMG) * _PERG,
        )
        nb = ng * 8
        start = startg * 8
        pltpu.sync_copy(idx_hbm.at[pl.ds(start, _MAXROWS), :], idx_v)

        def body(j, carry):
            g = start + j
            pltpu.async_copy(h_hbm.at[idx_v.at[j]], rows_v, sem).wait()
            pltpu.sync_copy(rows_v, out_hbm.at[pl.ds(g * _GB, _GB), :])
            return carry

        lax.fori_loop(0, nb, body, 0)

    return gather_kernel(h, idx2d)


def _to_col(v):
    ri = lax.broadcasted_iota(jnp.int32, (H, H), 0)
    ci = lax.broadcasted_iota(jnp.int32, (H, H), 1)
    bc = jnp.broadcast_to(v, (H, H))
    return jnp.sum(jnp.where(ri == ci, bc, 0.0), axis=1, keepdims=True)


def _exact_gelu(x):
    return 0.5 * x * (1.0 + lax.erf(x * (2.0 ** -0.5)))


_B = 1600
_K = E // _B


def _main_body(hi_ref, hj_ref, e_ref, wa_ref, we_ref, wb_ref,
               na_ref, ne_ref, nb_ref, batt_ref, bnode_ref,
               w_ref, ms_ref, g_ref):
    k = pl.program_id(0)

    @pl.when(k == 0)
    def _():
        ms_ref[...] = jnp.zeros((2, H), jnp.float32)
        g_ref[...] = jnp.zeros((H, ATT_EMB), jnp.float32)

    hi = hi_ref[...]
    hj = hj_ref[...]
    e = e_ref[...]

    pre_w = (
        jnp.dot(hi, wa_ref[...], preferred_element_type=jnp.float32)
        + jnp.dot(hj, wb_ref[...], preferred_element_type=jnp.float32)
        + jnp.dot(e, we_ref[...], preferred_element_type=jnp.float32)
        + batt_ref[...]
    )
    w = jnp.maximum(pre_w, 0.0) * INV_SQRT_DHEAD
    w_ref[...] = w

    pre_v = (
        jnp.dot(hi, na_ref[...], preferred_element_type=jnp.float32)
        + jnp.dot(hj, nb_ref[...], preferred_element_type=jnp.float32)
        + jnp.dot(e, ne_ref[...], preferred_element_type=jnp.float32)
        + bnode_ref[...]
    )
    vj = _exact_gelu(pre_v)

    m_prev = ms_ref[0:1, :]
    s_prev = ms_ref[1:2, :]
    mb = jnp.max(w, axis=0, keepdims=True)
    m_new = jnp.maximum(m_prev, mb)
    alpha = jnp.exp(m_prev - m_new)
    p = jnp.exp(w - m_new)
    ms_ref[0:1, :] = m_new
    ms_ref[1:2, :] = s_prev * alpha + jnp.sum(p, axis=0, keepdims=True)

    ptv = lax.dot_general(
        p, vj, (((0,), (0,)), ((), ())), preferred_element_type=jnp.float32
    )
    g_ref[...] = g_ref[...] * _to_col(alpha) + ptv


def _epilogue_body(g_ref, ms_ref, w1_ref, b1_ref, w2_ref, b2_ref, wh_ref,
                   out_ref):
    s_col = _to_col(ms_ref[1:2, :])
    gn = g_ref[...] / s_col
    ri = lax.broadcasted_iota(jnp.int32, (H, ATT_EMB), 0)
    ci = lax.broadcasted_iota(jnp.int32, (H, ATT_EMB), 1)
    gfea = jnp.sum(jnp.where(ci // D_HEAD == ri, gn, 0.0), axis=0,
                   keepdims=True)
    t = jnp.maximum(
        jnp.dot(gfea, w1_ref[...], preferred_element_type=jnp.float32)
        + b1_ref[...], 0.0)
    u = (jnp.dot(t, w2_ref[...], preferred_element_type=jnp.float32)
         + b2_ref[...])
    out_ref[...] = jnp.dot(u, wh_ref[...], preferred_element_type=jnp.float32)


_B2 = 6400


def _att_body(w_ref, ms_ref, att_ref):
    m = ms_ref[0:1, :]
    s = ms_ref[1:2, :]
    att_ref[...] = jnp.exp(w_ref[...] - m) / s


def kernel(h, edge_core, core_edge_idx, Watt, batt, Wnode, bnode,
           W1, b1, W2, b2, Wh):
    idx2d = core_edge_idx.reshape(_NB, _GB)
    idx2d = jnp.pad(idx2d, ((0, _NW), (0, 0)))
    gath = _sc_gather(h, idx2d)
    hi = gath[:E]
    hj = gath[E:]

    wa_t = Watt[:, :D_NODE].T
    we_t = Watt[:, D_NODE:D_NODE + D_EDGE].T
    wb_t = Watt[:, D_NODE + D_EDGE:].T
    na_t = Wnode[:, :D_NODE].T
    ne_t = Wnode[:, D_NODE:D_NODE + D_EDGE].T
    nb_t = Wnode[:, D_NODE + D_EDGE:].T
    batt2 = batt.reshape(1, H)
    bnode2 = bnode.reshape(1, ATT_EMB)

    full = lambda shape: pl.BlockSpec(shape, lambda k: (0, 0))
    w_logits, ms, g_acc = pl.pallas_call(
        _main_body,
        grid=(_K,),
        in_specs=[
            pl.BlockSpec((_B, D_NODE), lambda k: (k, 0)),
            pl.BlockSpec((_B, D_NODE), lambda k: (k, 0)),
            pl.BlockSpec((_B, D_EDGE), lambda k: (k, 0)),
            full((D_NODE, H)), full((D_EDGE, H)), full((D_NODE, H)),
            full((D_NODE, ATT_EMB)), full((D_EDGE, ATT_EMB)),
            full((D_NODE, ATT_EMB)),
            full((1, H)), full((1, ATT_EMB)),
        ],
        out_specs=[
            pl.BlockSpec((_B, H), lambda k: (k, 0)),
            full((2, H)),
            full((H, ATT_EMB)),
        ],
        out_shape=[
            jax.ShapeDtypeStruct((E, H), jnp.float32),
            jax.ShapeDtypeStruct((2, H), jnp.float32),
            jax.ShapeDtypeStruct((H, ATT_EMB), jnp.float32),
        ],
    )(hi, hj, edge_core, wa_t, we_t, wb_t, na_t, ne_t, nb_t, batt2, bnode2)

    global_fea = pl.pallas_call(
        _epilogue_body,
        out_shape=jax.ShapeDtypeStruct((1, ATT_EMB), jnp.float32),
    )(g_acc, ms, W1.T, b1.reshape(1, D_FF), W2.T, b2.reshape(1, ATT_EMB),
      Wh.T)

    att = pl.pallas_call(
        _att_body,
        grid=(E // _B2,),
        in_specs=[pl.BlockSpec((_B2, H), lambda k: (k, 0)), full((2, H))],
        out_specs=pl.BlockSpec((_B2, H), lambda k: (k, 0)),
        out_shape=jax.ShapeDtypeStruct((E, H), jnp.float32),
    )(w_logits, ms)

    return (h, edge_core, global_fea.reshape(ATT_EMB), att)


# speedup vs baseline: 5.1902x; 1.3475x over previous
import functools

import jax
import jax.numpy as jnp
from jax import lax
from jax.experimental import pallas as pl
from jax.experimental.pallas import tpu as pltpu
from jax.experimental.pallas import tpu_sc as plsc

N = 10000
E = 320000
D_NODE = 128
D_EDGE = 16
H = 20
ATT_EMB = 1280
D_HEAD = ATT_EMB // H
DIN = 2 * D_NODE + D_EDGE
D_FF = ATT_EMB * 4
INV_SQRT_DHEAD = 1.0 / (D_HEAD ** 0.5)

_C = 5
_EC = E // _C

_GB = 128
_NBC = (2 * _EC) // _GB
_NW = 32
_NGC = _NBC // 8
_PERG = _NGC // _NW
_REMG = _NGC - _PERG * _NW
_CUT = _NW - _REMG
_MAXROWS = (_PERG + 1) * 8
_CHB = 2
_CHROWS = _CHB * _GB


def _sc_gather(h, idx2d):
    mesh = plsc.VectorSubcoreMesh(core_axis_name="c", subcore_axis_name="s")

    @functools.partial(
        pl.kernel,
        mesh=mesh,
        out_type=jax.ShapeDtypeStruct((2 * _EC, D_NODE), jnp.float32),
        scratch_types=[
            pltpu.VMEM((_MAXROWS, _GB), jnp.int32),
            pltpu.VMEM((_CHROWS, D_NODE), jnp.float32),
            pltpu.VMEM((_CHROWS, D_NODE), jnp.float32),
            pltpu.SemaphoreType.DMA,
            pltpu.SemaphoreType.DMA,
        ],
    )
    def gather_kernel(h_hbm, idx_hbm, out_hbm, idx_v, buf_a, buf_b,
                      sem_a, sem_b):
        c = lax.axis_index("c")
        s = lax.axis_index("s")
        wid = s * 2 + c
        extra = jnp.maximum(wid - _CUT, 0)
        ng = jnp.where(wid >= _CUT, _PERG + 1, _PERG)
        start = (wid * _PERG + extra) * 8
        nchunk = ng * (8 // _CHB)
        pltpu.sync_copy(idx_hbm.at[pl.ds(start, _MAXROWS), :], idx_v)

        def fire(buf, sem, t):
            cps = []
            for i in range(_CHB):
                cps.append(pltpu.async_copy(
                    h_hbm.at[idx_v.at[t * _CHB + i]],
                    buf.at[pl.ds(i * _GB, _GB)], sem))
            return cps

        def drain(cps):
            for cp in cps:
                cp.wait()

        def write(buf, t):
            pltpu.sync_copy(
                buf, out_hbm.at[pl.ds((start + t * _CHB) * _GB, _CHROWS)])

        def drain_a():
            for i in range(_CHB):
                pltpu.make_async_copy(
                    h_hbm.at[idx_v.at[0]],
                    buf_a.at[pl.ds(i * _GB, _GB)], sem_a).wait()

        fire(buf_a, sem_a, 0)

        def body2(t2, carry):
            ta = 2 * t2
            cps_b = fire(buf_b, sem_b, ta + 1)
            drain_a()
            write(buf_a, ta)
            fire(buf_a, sem_a, jnp.minimum(ta + 2, nchunk - 1))
            drain(cps_b)
            write(buf_b, ta + 1)
            return carry

        lax.fori_loop(0, nchunk // 2, body2, 0)
        drain_a()

    return gather_kernel(h, idx2d)


def _to_col(v):
    ri = lax.broadcasted_iota(jnp.int32, (H, H), 0)
    ci = lax.broadcasted_iota(jnp.int32, (H, H), 1)
    bc = jnp.broadcast_to(v, (H, H))
    return jnp.sum(jnp.where(ri == ci, bc, 0.0), axis=1, keepdims=True)


def _exact_gelu(x):
    return 0.5 * x * (1.0 + lax.erf(x * (2.0 ** -0.5)))


_B = 1600
_KC = _EC // _B


def _main_body(hi_ref, hj_ref, e_ref, watt_ref, wnode_ref,
               batt_ref, bnode_ref, w_ref, ms_ref, g_ref):
    k = pl.program_id(0)

    @pl.when(k == 0)
    def _():
        ms_ref[...] = jnp.zeros((2, H), jnp.float32)
        g_ref[...] = jnp.zeros((H, ATT_EMB), jnp.float32)

    x = jnp.concatenate(
        [hi_ref[...].astype(jnp.bfloat16),
         e_ref[...].astype(jnp.bfloat16),
         hj_ref[...].astype(jnp.bfloat16)], axis=1)

    pre_w = (jnp.dot(x, watt_ref[...], preferred_element_type=jnp.float32)
             + batt_ref[...])
    w = jnp.maximum(pre_w, 0.0) * INV_SQRT_DHEAD
    w_ref[...] = w

    pre_v = (jnp.dot(x, wnode_ref[...], preferred_element_type=jnp.float32)
             + bnode_ref[...])
    vj = _exact_gelu(pre_v)

    m_prev = ms_ref[0:1, :]
    s_prev = ms_ref[1:2, :]
    mb = jnp.max(w, axis=0, keepdims=True)
    m_new = jnp.maximum(m_prev, mb)
    alpha = jnp.exp(m_prev - m_new)
    p = jnp.exp(w - m_new)
    ms_ref[0:1, :] = m_new
    ms_ref[1:2, :] = s_prev * alpha + jnp.sum(p, axis=0, keepdims=True)

    ptv = lax.dot_general(
        p.astype(jnp.bfloat16), vj.astype(jnp.bfloat16),
        (((0,), (0,)), ((), ())), preferred_element_type=jnp.float32
    )
    g_ref[...] = g_ref[...] * _to_col(alpha) + ptv


def _epilogue_body(g_ref, ms_ref, w1_ref, b1_ref, w2_ref, b2_ref, wh_ref,
                   out_ref, msg_ref):
    m = ms_ref[0:1, :]
    for c in range(1, _C):
        m = jnp.maximum(m, ms_ref[2 * c:2 * c + 1, :])
    s = jnp.zeros((1, H), jnp.float32)
    g = jnp.zeros((H, ATT_EMB), jnp.float32)
    for c in range(_C):
        a_c = jnp.exp(ms_ref[2 * c:2 * c + 1, :] - m)
        s = s + a_c * ms_ref[2 * c + 1:2 * c + 2, :]
        g = g + _to_col(a_c) * g_ref[H * c:H * (c + 1), :]
    msg_ref[0:1, :] = m
    msg_ref[1:2, :] = s

    gn = g / _to_col(s)
    ri = lax.broadcasted_iota(jnp.int32, (H, ATT_EMB), 0)
    ci = lax.broadcasted_iota(jnp.int32, (H, ATT_EMB), 1)
    gfea = jnp.sum(jnp.where(ci // D_HEAD == ri, gn, 0.0), axis=0,
                   keepdims=True)
    t = jnp.maximum(
        jnp.dot(gfea.astype(jnp.bfloat16), w1_ref[...],
                preferred_element_type=jnp.float32) + b1_ref[...], 0.0)
    u = (jnp.dot(t.astype(jnp.bfloat16), w2_ref[...],
                 preferred_element_type=jnp.float32) + b2_ref[...])
    out_ref[...] = jnp.dot(u.astype(jnp.bfloat16), wh_ref[...],
                           preferred_element_type=jnp.float32)


_B2 = 6400


def _att_body(w_ref, ms_ref, att_ref):
    m = ms_ref[0:1, :]
    s = ms_ref[1:2, :]
    att_ref[...] = jnp.exp(w_ref[...] - m) / s


def kernel(h, edge_core, core_edge_idx, Watt, batt, Wnode, bnode,
           W1, b1, W2, b2, Wh):
    bf16 = jnp.bfloat16
    nb_half = E // _GB
    nbc_half = _EC // _GB
    idx2d = core_edge_idx.reshape(2 * nb_half, _GB)

    watt_t = Watt.T.astype(bf16)
    wnode_t = Wnode.T.astype(bf16)
    batt2 = batt.reshape(1, H)
    bnode2 = bnode.reshape(1, ATT_EMB)

    full = lambda shape: pl.BlockSpec(shape, lambda k: (0, 0))
    main_call = pl.pallas_call(
        _main_body,
        grid=(_KC,),
        in_specs=[
            pl.BlockSpec((_B, D_NODE), lambda k: (k, 0)),
            pl.BlockSpec((_B, D_NODE), lambda k: (k + _KC, 0)),
            pl.BlockSpec((_B, D_EDGE), lambda k: (k, 0)),
            full((DIN, H)), full((DIN, ATT_EMB)),
            full((1, H)), full((1, ATT_EMB)),
        ],
        out_specs=[
            pl.BlockSpec((_B, H), lambda k: (k, 0)),
            full((2, H)),
            full((H, ATT_EMB)),
        ],
        out_shape=[
            jax.ShapeDtypeStruct((_EC, H), jnp.float32),
            jax.ShapeDtypeStruct((2, H), jnp.float32),
            jax.ShapeDtypeStruct((H, ATT_EMB), jnp.float32),
        ],
    )

    w_chunks, ms_chunks, g_chunks = [], [], []
    for c in range(_C):
        idx_c = jnp.concatenate(
            [idx2d[c * nbc_half:(c + 1) * nbc_half],
             idx2d[nb_half + c * nbc_half:nb_half + (c + 1) * nbc_half]],
            axis=0)
        gath = _sc_gather(h, idx_c)
        edge_c = lax.slice_in_dim(edge_core, c * _EC, (c + 1) * _EC)
        w_c, ms_c, g_c = main_call(gath, gath, edge_c, watt_t, wnode_t,
                                   batt2, bnode2)
        w_chunks.append(w_c)
        ms_chunks.append(ms_c)
        g_chunks.append(g_c)

    ms_all = jnp.concatenate(ms_chunks, axis=0)
    g_all = jnp.concatenate(g_chunks, axis=0)

    global_fea, ms_g = pl.pallas_call(
        _epilogue_body,
        out_shape=[
            jax.ShapeDtypeStruct((1, ATT_EMB), jnp.float32),
            jax.ShapeDtypeStruct((2, H), jnp.float32),
        ],
    )(g_all, ms_all, W1.T.astype(bf16), b1.reshape(1, D_FF),
      W2.T.astype(bf16), b2.reshape(1, ATT_EMB), Wh.T.astype(bf16))

    att_call = pl.pallas_call(
        _att_body,
        grid=(_EC // _B2,),
        in_specs=[pl.BlockSpec((_B2, H), lambda k: (k, 0)), full((2, H))],
        out_specs=pl.BlockSpec((_B2, H), lambda k: (k, 0)),
        out_shape=jax.ShapeDtypeStruct((_EC, H), jnp.float32),
    )
    att = jnp.concatenate([att_call(w_c, ms_g) for w_c in w_chunks],
                          axis=0)

    return (h, edge_core, global_fea.reshape(ATT_EMB), att)


# speedup vs baseline: 5.4022x; 1.0408x over previous
import functools

import jax
import jax.numpy as jnp
from jax import lax
from jax.experimental import pallas as pl
from jax.experimental.pallas import tpu as pltpu
from jax.experimental.pallas import tpu_sc as plsc

N = 10000
E = 320000
D_NODE = 128
D_EDGE = 16
H = 20
ATT_EMB = 1280
D_HEAD = ATT_EMB // H
DIN = 2 * D_NODE + D_EDGE
D_FF = ATT_EMB * 4
INV_SQRT_DHEAD = 1.0 / (D_HEAD ** 0.5)

_C = 1
_EC = E // _C
_B = 1280
_KC = _EC // _B

_GB = 128
_NBB = _B // _GB
_NBC = (2 * _EC) // _GB
_NHALF = _NBC // 2
_NW = 32
_NGC = _NBC // 8
_PERG = _NGC // _NW
_REMG = _NGC - _PERG * _NW
_CUT = _NW - _REMG
_MAXROWS = (_PERG + 1) * 8
_CHB = 2
_CHROWS = _CHB * _GB


def _out_row(gb):
    half = (gb >= _NHALF).astype(jnp.int32)
    q = gb - half * _NHALF
    blk = q // _NBB
    off = q % _NBB
    return blk * (2 * _B) + half * _B + off * _GB


def _sc_gather(h, idx2d):
    mesh = plsc.VectorSubcoreMesh(core_axis_name="c", subcore_axis_name="s")

    @functools.partial(
        pl.kernel,
        mesh=mesh,
        out_type=jax.ShapeDtypeStruct((2 * _EC, D_NODE), jnp.float32),
        scratch_types=[
            pltpu.VMEM((_MAXROWS, _GB), jnp.int32),
            pltpu.VMEM((_CHROWS, D_NODE), jnp.float32),
            pltpu.VMEM((_CHROWS, D_NODE), jnp.float32),
            pltpu.SemaphoreType.DMA,
            pltpu.SemaphoreType.DMA,
        ],
    )
    def gather_kernel(h_hbm, idx_hbm, out_hbm, idx_v, buf_a, buf_b,
                      sem_a, sem_b):
        c = lax.axis_index("c")
        s = lax.axis_index("s")
        wid = s * 2 + c
        extra = jnp.maximum(wid - _CUT, 0)
        ng = jnp.where(wid >= _CUT, _PERG + 1, _PERG)
        start = (wid * _PERG + extra) * 8
        nchunk = ng * (8 // _CHB)
        pltpu.sync_copy(idx_hbm.at[pl.ds(start, _MAXROWS), :], idx_v)

        def fire(buf, sem, t):
            cps = []
            for i in range(_CHB):
                cps.append(pltpu.async_copy(
                    h_hbm.at[idx_v.at[t * _CHB + i]],
                    buf.at[pl.ds(i * _GB, _GB)], sem))
            return cps

        def drain(cps):
            for cp in cps:
                cp.wait()

        def write(buf, t):
            for i in range(_CHB):
                gb = start + t * _CHB + i
                pltpu.sync_copy(
                    buf.at[pl.ds(i * _GB, _GB)],
                    out_hbm.at[pl.ds(_out_row(gb), _GB)])

        def drain_a():
            for i in range(_CHB):
                pltpu.make_async_copy(
                    h_hbm.at[idx_v.at[0]],
                    buf_a.at[pl.ds(i * _GB, _GB)], sem_a).wait()

        fire(buf_a, sem_a, 0)

        def body2(t2, carry):
            ta = 2 * t2
            cps_b = fire(buf_b, sem_b, ta + 1)
            drain_a()
            write(buf_a, ta)
            fire(buf_a, sem_a, jnp.minimum(ta + 2, nchunk - 1))
            drain(cps_b)
            write(buf_b, ta + 1)
            return carry

        lax.fori_loop(0, nchunk // 2, body2, 0)
        drain_a()

    return gather_kernel(h, idx2d)


def _to_col(v):
    ri = lax.broadcasted_iota(jnp.int32, (H, H), 0)
    ci = lax.broadcasted_iota(jnp.int32, (H, H), 1)
    bc = jnp.broadcast_to(v, (H, H))
    return jnp.sum(jnp.where(ri == ci, bc, 0.0), axis=1, keepdims=True)


def _exact_gelu(x):
    return 0.5 * x * (1.0 + lax.erf(x * (2.0 ** -0.5)))


def _main_body(hij_ref, e_ref, watt_ref, wnode_ref,
               batt_ref, bnode_ref, w_ref, ms_ref, g_ref):
    k = pl.program_id(0)

    @pl.when(k == 0)
    def _():
        ms_ref[...] = jnp.zeros((2, H), jnp.float32)
        g_ref[...] = jnp.zeros((H, ATT_EMB), jnp.float32)

    x = jnp.concatenate(
        [hij_ref[0:_B, :].astype(jnp.bfloat16),
         e_ref[...].astype(jnp.bfloat16),
         hij_ref[_B:2 * _B, :].astype(jnp.bfloat16)], axis=1)

    pre_w = (jnp.dot(x, watt_ref[...], preferred_element_type=jnp.float32)
             + batt_ref[...])
    w = jnp.maximum(pre_w, 0.0) * INV_SQRT_DHEAD
    w_ref[...] = w

    pre_v = (jnp.dot(x, wnode_ref[...], preferred_element_type=jnp.float32)
             + bnode_ref[...])
    vj = _exact_gelu(pre_v)

    m_prev = ms_ref[0:1, :]
    s_prev = ms_ref[1:2, :]
    mb = jnp.max(w, axis=0, keepdims=True)
    m_new = jnp.maximum(m_prev, mb)
    alpha = jnp.exp(m_prev - m_new)
    p = jnp.exp(w - m_new)
    ms_ref[0:1, :] = m_new
    ms_ref[1:2, :] = s_prev * alpha + jnp.sum(p, axis=0, keepdims=True)

    ptv = lax.dot_general(
        p.astype(jnp.bfloat16), vj.astype(jnp.bfloat16),
        (((0,), (0,)), ((), ())), preferred_element_type=jnp.float32
    )
    g_ref[...] = g_ref[...] * _to_col(alpha) + ptv


_FFB = 640
_KF = D_FF // _FFB


def _epilogue_body(g_ref, ms_ref, w1_ref, b1_ref, w2_ref, b2_ref, wh_ref,
                   out_ref, msg_ref, gfea_ref, uacc_ref):
    i = pl.program_id(0)

    @pl.when(i == 0)
    def _():
        m = ms_ref[0:1, :]
        for c in range(1, _C):
            m = jnp.maximum(m, ms_ref[2 * c:2 * c + 1, :])
        s = jnp.zeros((1, H), jnp.float32)
        g = jnp.zeros((H, ATT_EMB), jnp.float32)
        for c in range(_C):
            a_c = jnp.exp(ms_ref[2 * c:2 * c + 1, :] - m)
            s = s + a_c * ms_ref[2 * c + 1:2 * c + 2, :]
            g = g + _to_col(a_c) * g_ref[H * c:H * (c + 1), :]
        msg_ref[0:1, :] = m
        msg_ref[1:2, :] = s

        gn = g / _to_col(s)
        ri = lax.broadcasted_iota(jnp.int32, (H, ATT_EMB), 0)
        ci = lax.broadcasted_iota(jnp.int32, (H, ATT_EMB), 1)
        gfea_ref[...] = jnp.sum(jnp.where(ci // D_HEAD == ri, gn, 0.0),
                                axis=0, keepdims=True)
        uacc_ref[...] = jnp.zeros((1, ATT_EMB), jnp.float32)

    t = jnp.maximum(
        lax.dot_general(gfea_ref[...].astype(jnp.bfloat16), w1_ref[...],
                        (((1,), (1,)), ((), ())),
                        preferred_element_type=jnp.float32)
        + b1_ref[...], 0.0)
    uacc_ref[...] += lax.dot_general(
        t.astype(jnp.bfloat16), w2_ref[...], (((1,), (1,)), ((), ())),
        preferred_element_type=jnp.float32)

    @pl.when(i == _KF - 1)
    def _():
        u = uacc_ref[...] + b2_ref[...]
        out_ref[...] = lax.dot_general(
            u.astype(jnp.bfloat16), wh_ref[...], (((1,), (1,)), ((), ())),
            preferred_element_type=jnp.float32)


_B2 = 6400


def _att_body(w_ref, ms_ref, att_ref):
    m = ms_ref[0:1, :]
    s = ms_ref[1:2, :]
    att_ref[...] = jnp.exp(w_ref[...] - m) / s


def kernel(h, edge_core, core_edge_idx, Watt, batt, Wnode, bnode,
           W1, b1, W2, b2, Wh):
    bf16 = jnp.bfloat16
    nb_half = E // _GB
    nbc_half = _EC // _GB
    idx2d = core_edge_idx.reshape(2 * nb_half, _GB)

    watt_t = Watt.T.astype(bf16)
    wnode_t = Wnode.T.astype(bf16)
    batt2 = batt.reshape(1, H)
    bnode2 = bnode.reshape(1, ATT_EMB)

    full = lambda shape: pl.BlockSpec(shape, lambda k: (0, 0))
    main_call = pl.pallas_call(
        _main_body,
        grid=(_KC,),
        in_specs=[
            pl.BlockSpec((2 * _B, D_NODE), lambda k: (k, 0)),
            pl.BlockSpec((_B, D_EDGE), lambda k: (k, 0)),
            full((DIN, H)), full((DIN, ATT_EMB)),
            full((1, H)), full((1, ATT_EMB)),
        ],
        out_specs=[
            pl.BlockSpec((_B, H), lambda k: (k, 0)),
            full((2, H)),
            full((H, ATT_EMB)),
        ],
        out_shape=[
            jax.ShapeDtypeStruct((_EC, H), jnp.float32),
            jax.ShapeDtypeStruct((2, H), jnp.float32),
            jax.ShapeDtypeStruct((H, ATT_EMB), jnp.float32),
        ],
    )

    w_chunks, ms_chunks, g_chunks = [], [], []
    for c in range(_C):
        if _C == 1:
            idx_c = idx2d
        else:
            idx_c = jnp.concatenate(
                [idx2d[c * nbc_half:(c + 1) * nbc_half],
                 idx2d[nb_half + c * nbc_half:
                       nb_half + (c + 1) * nbc_half]], axis=0)
        gath = _sc_gather(h, idx_c)
        edge_c = (edge_core if _C == 1 else
                  lax.slice_in_dim(edge_core, c * _EC, (c + 1) * _EC))
        w_c, ms_c, g_c = main_call(gath, edge_c, watt_t, wnode_t,
                                   batt2, bnode2)
        w_chunks.append(w_c)
        ms_chunks.append(ms_c)
        g_chunks.append(g_c)

    ms_all = (ms_chunks[0] if _C == 1 else
              jnp.concatenate(ms_chunks, axis=0))
    g_all = (g_chunks[0] if _C == 1 else
             jnp.concatenate(g_chunks, axis=0))

    global_fea, ms_g = pl.pallas_call(
        _epilogue_body,
        grid=(_KF,),
        in_specs=[
            full((_C * H, ATT_EMB)), full((2 * _C, H)),
            pl.BlockSpec((_FFB, ATT_EMB), lambda i: (i, 0)),
            pl.BlockSpec((1, _FFB), lambda i: (0, i)),
            pl.BlockSpec((ATT_EMB, _FFB), lambda i: (0, i)),
            full((1, ATT_EMB)), full((ATT_EMB, ATT_EMB)),
        ],
        out_specs=[full((1, ATT_EMB)), full((2, H))],
        out_shape=[
            jax.ShapeDtypeStruct((1, ATT_EMB), jnp.float32),
            jax.ShapeDtypeStruct((2, H), jnp.float32),
        ],
        scratch_shapes=[
            pltpu.VMEM((1, ATT_EMB), jnp.float32),
            pltpu.VMEM((1, ATT_EMB), jnp.float32),
        ],
    )(g_all, ms_all, W1.astype(bf16), b1.reshape(1, D_FF),
      W2.astype(bf16), b2.reshape(1, ATT_EMB), Wh.astype(bf16))

    att_call = pl.pallas_call(
        _att_body,
        grid=(_EC // _B2,),
        in_specs=[pl.BlockSpec((_B2, H), lambda k: (k, 0)), full((2, H))],
        out_specs=pl.BlockSpec((_B2, H), lambda k: (k, 0)),
        out_shape=jax.ShapeDtypeStruct((_EC, H), jnp.float32),
    )
    att_chunks = [att_call(w_c, ms_g) for w_c in w_chunks]
    att = (att_chunks[0] if _C == 1 else
           jnp.concatenate(att_chunks, axis=0))

    return (h, edge_core, global_fea.reshape(ATT_EMB), att)
